# Initial kernel scaffold; baseline (speedup 1.0000x reference)
import jax
import jax.numpy as jnp
from jax.experimental import pallas as pl


def kernel(x_gene, x_cell, W1_gg, b1_gg, W1_rev, b1_rev, W1_cc, b1_cc, W2_gg, b2_gg, W2_rev, b2_rev, W2_cc, b2_cc, edge_index_gg, edge_index_gg_rev, edge_index_cc, edge_label_index):
    raise NotImplementedError("write your pallas kernel here")


# speedup vs baseline: 7.7298x; 7.7298x over previous
import functools

import jax
import jax.numpy as jnp
from jax import lax
from jax.experimental import pallas as pl
from jax.experimental.pallas import tpu as pltpu
from jax.experimental.pallas import tpu_sc as plsc

N = 10000
N_PAD = 10240
D = 128
H1 = 256
H2 = 128
E = 320000
E_LBL = 100000
E_LBL_PAD = 102400

NC = 2
NS = 16
CH = 80
RPT = N_PAD // NS
EPT = E // NS
LPT = E_LBL_PAD // NS

_MESH = plsc.VectorSubcoreMesh(core_axis_name="c", subcore_axis_name="s")


def _f32(shape):
    return jax.ShapeDtypeStruct(shape, jnp.float32)


@functools.partial(
    pl.kernel,
    out_type=[_f32((N_PAD, 16)), _f32((N_PAD, 16))],
    mesh=_MESH,
    scratch_types=[
        pltpu.VMEM((CH,), jnp.int32),
        pltpu.VMEM((CH, 16), jnp.float32),
        pltpu.VMEM_SHARED((N_PAD, 16), jnp.float32),
    ],
)
def _sc_degree(dst_gg, dst_rev, ones_hbm, deg_gg, deg_rev, idx_v, ones_v, acc_sh):
    cid = lax.axis_index("c")
    sid = lax.axis_index("s")
    r0 = sid * RPT
    pltpu.sync_copy(ones_hbm.at[pl.ds(r0, RPT)], acc_sh.at[pl.ds(r0, RPT)])
    pltpu.sync_copy(ones_hbm.at[pl.ds(0, CH)], ones_v)
    plsc.subcore_barrier()

    def run(dst_hbm):
        def body(i, carry):
            off = sid * EPT + i * CH
            pltpu.sync_copy(dst_hbm.at[pl.ds(off, CH)], idx_v)
            pltpu.sync_copy(ones_v, acc_sh.at[idx_v], add=True)
            return carry
        lax.fori_loop(0, EPT // CH, body, 0)

    @pl.when(cid == 0)
    def _():
        run(dst_gg)

    @pl.when(cid == 1)
    def _():
        run(dst_rev)

    plsc.subcore_barrier()

    @pl.when(cid == 0)
    def _():
        pltpu.sync_copy(acc_sh.at[pl.ds(r0, RPT)], deg_gg.at[pl.ds(r0, RPT)])

    @pl.when(cid == 1)
    def _():
        pltpu.sync_copy(acc_sh.at[pl.ds(r0, RPT)], deg_rev.at[pl.ds(r0, RPT)])


def _make_sc_scatter(width):
    @functools.partial(
        pl.kernel,
        out_type=[_f32((N_PAD, width))] * 4,
        mesh=_MESH,
        scratch_types=[
            pltpu.VMEM((CH,), jnp.int32),
            pltpu.VMEM((CH,), jnp.int32),
            pltpu.VMEM((CH, width), jnp.float32),
            pltpu.VMEM_SHARED((N_PAD, width), jnp.float32),
            pltpu.SemaphoreType.DMA,
        ],
    )
    def scatter(y_gg_a, y_gg_b, y_rev_a, y_rev_b,
                src_gg, dst_gg, src_rev, dst_rev,
                acc_gg_a, acc_gg_b, acc_rev_a, acc_rev_b,
                sidx_v, didx_v, rows_v, acc_sh, sem):
        cid = lax.axis_index("c")
        sid = lax.axis_index("s")
        r0 = sid * RPT

        def run(y_hbm, src_hbm, dst_hbm, out_hbm):
            pltpu.sync_copy(y_hbm.at[pl.ds(r0, RPT)], acc_sh.at[pl.ds(r0, RPT)])
            plsc.subcore_barrier()

            def body(i, carry):
                off = sid * EPT + i * CH
                pltpu.sync_copy(src_hbm.at[pl.ds(off, CH)], sidx_v)
                pltpu.sync_copy(dst_hbm.at[pl.ds(off, CH)], didx_v)
                pltpu.async_copy(y_hbm.at[sidx_v], rows_v, sem).wait()
                pltpu.sync_copy(rows_v, acc_sh.at[didx_v], add=True)
                return carry
            lax.fori_loop(0, EPT // CH, body, 0)
            plsc.subcore_barrier()
            pltpu.sync_copy(acc_sh.at[pl.ds(r0, RPT)], out_hbm.at[pl.ds(r0, RPT)])
            plsc.subcore_barrier()

        @pl.when(cid == 0)
        def _():
            run(y_gg_a, src_gg, dst_gg, acc_gg_a)
            run(y_rev_a, src_rev, dst_rev, acc_rev_a)

        @pl.when(cid == 1)
        def _():
            run(y_gg_b, src_gg, dst_gg, acc_gg_b)
            run(y_rev_b, src_rev, dst_rev, acc_rev_b)

    return scatter


_sc_scatter_l1 = _make_sc_scatter(H1 // 2)

EPT2 = E // (NC * NS)


@functools.partial(
    pl.kernel,
    out_type=[_f32((N_PAD, H2))] * 4,
    mesh=_MESH,
    scratch_types=[
        pltpu.VMEM((CH,), jnp.int32),
        pltpu.VMEM((CH,), jnp.int32),
        pltpu.VMEM((CH, H2), jnp.float32),
        pltpu.VMEM_SHARED((N_PAD, H2), jnp.float32),
        pltpu.SemaphoreType.DMA,
    ],
)
def _sc_scatter_l2(y_gg, y_rev, zeros_hbm, src_gg, dst_gg, src_rev, dst_rev,
                   p_gg_0, p_gg_1, p_rev_0, p_rev_1,
                   sidx_v, didx_v, rows_v, acc_sh, sem):
    cid = lax.axis_index("c")
    sid = lax.axis_index("s")
    r0 = sid * RPT

    def run(y_hbm, init_hbm, src_hbm, dst_hbm, out_hbm):
        pltpu.sync_copy(init_hbm.at[pl.ds(r0, RPT)], acc_sh.at[pl.ds(r0, RPT)])
        plsc.subcore_barrier()

        def body(i, carry):
            off = cid * (E // 2) + sid * EPT2 + i * CH
            pltpu.sync_copy(src_hbm.at[pl.ds(off, CH)], sidx_v)
            pltpu.sync_copy(dst_hbm.at[pl.ds(off, CH)], didx_v)
            pltpu.async_copy(y_hbm.at[sidx_v], rows_v, sem).wait()
            pltpu.sync_copy(rows_v, acc_sh.at[didx_v], add=True)
            return carry
        lax.fori_loop(0, EPT2 // CH, body, 0)
        plsc.subcore_barrier()
        pltpu.sync_copy(acc_sh.at[pl.ds(r0, RPT)], out_hbm.at[pl.ds(r0, RPT)])
        plsc.subcore_barrier()

    @pl.when(cid == 0)
    def _():
        run(y_gg, y_gg, src_gg, dst_gg, p_gg_0)
        run(y_rev, y_rev, src_rev, dst_rev, p_rev_0)

    @pl.when(cid == 1)
    def _():
        run(y_gg, zeros_hbm, src_gg, dst_gg, p_gg_1)
        run(y_rev, zeros_hbm, src_rev, dst_rev, p_rev_1)


@functools.partial(
    pl.kernel,
    out_type=[_f32((E_LBL_PAD, D)), _f32((E_LBL_PAD, D))],
    mesh=_MESH,
    scratch_types=[
        pltpu.VMEM((CH,), jnp.int32),
        pltpu.VMEM((CH, D), jnp.float32),
        pltpu.SemaphoreType.DMA,
    ],
)
def _sc_gather_lbl(g2, i0, i1, ef1, ef2, idx_v, rows_v, sem):
    cid = lax.axis_index("c")
    sid = lax.axis_index("s")

    def run(idx_hbm, out_hbm):
        def body(i, carry):
            off = sid * LPT + i * CH
            pltpu.sync_copy(idx_hbm.at[pl.ds(off, CH)], idx_v)
            pltpu.async_copy(g2.at[idx_v], rows_v, sem).wait()
            pltpu.sync_copy(rows_v, out_hbm.at[pl.ds(off, CH)])
            return carry
        lax.fori_loop(0, LPT // CH, body, 0)

    @pl.when(cid == 0)
    def _():
        run(i0, ef1)

    @pl.when(cid == 1)
    def _():
        run(i1, ef2)


BM = 1024


def _dis(deg_block):
    return lax.rsqrt(deg_block[:, :1])


def _tc_stage1_body(x_ref, wgg_ref, wrev_ref, dgg_ref, drev_ref,
                    ya_gg_ref, yb_gg_ref, ya_rev_ref, yb_rev_ref):
    x = x_ref[...]
    dis_gg = _dis(dgg_ref[...])
    dis_rev = _dis(drev_ref[...])
    y = jnp.dot(x, wgg_ref[...], preferred_element_type=jnp.float32) * dis_gg
    ya_gg_ref[...] = y[:, : H1 // 2]
    yb_gg_ref[...] = y[:, H1 // 2:]
    y = jnp.dot(x, wrev_ref[...], preferred_element_type=jnp.float32) * dis_rev
    ya_rev_ref[...] = y[:, : H1 // 2]
    yb_rev_ref[...] = y[:, H1 // 2:]


def _tc_stage1(x, wgg, wrev, deg_gg, deg_rev):
    row = lambda i: (i, 0)
    full = lambda i: (0, 0)
    return pl.pallas_call(
        _tc_stage1_body,
        grid=(N_PAD // BM,),
        in_specs=[
            pl.BlockSpec((BM, D), row),
            pl.BlockSpec((D, H1), full),
            pl.BlockSpec((D, H1), full),
            pl.BlockSpec((BM, 16), row),
            pl.BlockSpec((BM, 16), row),
        ],
        out_specs=[pl.BlockSpec((BM, H1 // 2), row)] * 4,
        out_shape=[_f32((N_PAD, H1 // 2))] * 4,
    )(x, wgg, wrev, deg_gg, deg_rev)


def _tc_stage2_body(agg_a_ref, agg_b_ref, arev_a_ref, arev_b_ref,
                    dgg_ref, drev_ref, b1gg_ref, b1rev_ref,
                    w2gg_ref, w2rev_ref,
                    ya_gg_ref, ya_rev_ref):
    dis_gg = _dis(dgg_ref[...])
    dis_rev = _dis(drev_ref[...])
    b1gg = b1gg_ref[...]
    b1rev = b1rev_ref[...]
    h = H1 // 2
    ga = agg_a_ref[...] * dis_gg + b1gg[:, :h] + arev_a_ref[...] * dis_rev + b1rev[:, :h]
    gb = agg_b_ref[...] * dis_gg + b1gg[:, h:] + arev_b_ref[...] * dis_rev + b1rev[:, h:]
    g = jnp.concatenate([jnp.maximum(ga, 0.0), jnp.maximum(gb, 0.0)], axis=1)
    ya_gg_ref[...] = jnp.dot(g, w2gg_ref[...], preferred_element_type=jnp.float32) * dis_gg
    ya_rev_ref[...] = jnp.dot(g, w2rev_ref[...], preferred_element_type=jnp.float32) * dis_rev


def _tc_stage2(acc, deg_gg, deg_rev, b1gg, b1rev, w2gg, w2rev):
    row = lambda i: (i, 0)
    full = lambda i: (0, 0)
    return pl.pallas_call(
        _tc_stage2_body,
        grid=(N_PAD // BM,),
        in_specs=[
            pl.BlockSpec((BM, H1 // 2), row)] * 4 + [
            pl.BlockSpec((BM, 16), row),
            pl.BlockSpec((BM, 16), row),
            pl.BlockSpec((1, H1), full),
            pl.BlockSpec((1, H1), full),
            pl.BlockSpec((H1, H2), full),
            pl.BlockSpec((H1, H2), full),
        ],
        out_specs=[pl.BlockSpec((BM, H2), row)] * 2,
        out_shape=[_f32((N_PAD, H2))] * 2,
    )(*acc, deg_gg, deg_rev, b1gg, b1rev, w2gg, w2rev)


def _tc_stage3_body(pgg0_ref, pgg1_ref, prev0_ref, prev1_ref,
                    dgg_ref, drev_ref, b2gg_ref, b2rev_ref, g2_ref):
    dis_gg = _dis(dgg_ref[...])
    dis_rev = _dis(drev_ref[...])
    gg = pgg0_ref[...] + pgg1_ref[...]
    rv = prev0_ref[...] + prev1_ref[...]
    g2_ref[...] = gg * dis_gg + b2gg_ref[...] + rv * dis_rev + b2rev_ref[...]


def _tc_stage3(acc, deg_gg, deg_rev, b2gg, b2rev):
    row = lambda i: (i, 0)
    full = lambda i: (0, 0)
    return pl.pallas_call(
        _tc_stage3_body,
        grid=(N_PAD // BM,),
        in_specs=[
            pl.BlockSpec((BM, H2), row)] * 4 + [
            pl.BlockSpec((BM, 16), row),
            pl.BlockSpec((BM, 16), row),
            pl.BlockSpec((1, H2), full),
            pl.BlockSpec((1, H2), full),
        ],
        out_specs=pl.BlockSpec((BM, H2), row),
        out_shape=_f32((N_PAD, H2)),
    )(*acc, deg_gg, deg_rev, b2gg, b2rev)


def _tc_dot_body(a_ref, b_ref, o_ref):
    o_ref[...] = jnp.sum(a_ref[...] * b_ref[...], axis=1, keepdims=True)


def _tc_dot(ef1, ef2):
    bm = 1024
    row = lambda i: (i, 0)
    return pl.pallas_call(
        _tc_dot_body,
        grid=(E_LBL_PAD // bm,),
        in_specs=[pl.BlockSpec((bm, D), row)] * 2,
        out_specs=pl.BlockSpec((bm, 1), row),
        out_shape=_f32((E_LBL_PAD, 1)),
    )(ef1, ef2)


def kernel(x_gene, x_cell, W1_gg, b1_gg, W1_rev, b1_rev, W1_cc, b1_cc,
           W2_gg, b2_gg, W2_rev, b2_rev, W2_cc, b2_cc,
           edge_index_gg, edge_index_gg_rev, edge_index_cc, edge_label_index):
    src_gg, dst_gg = edge_index_gg[0], edge_index_gg[1]
    src_rev, dst_rev = edge_index_gg_rev[0], edge_index_gg_rev[1]
    pad = jnp.zeros((E_LBL_PAD - E_LBL,), jnp.int32)
    i0 = jnp.concatenate([edge_label_index[0], pad])
    i1 = jnp.concatenate([edge_label_index[1], pad])
    ones16 = jnp.ones((N_PAD, 16), jnp.float32)
    xg = jnp.pad(x_gene, ((0, N_PAD - N), (0, 0)))

    deg_gg, deg_rev = _sc_degree(dst_gg, dst_rev, ones16)

    y1 = _tc_stage1(xg, W1_gg, W1_rev, deg_gg, deg_rev)
    acc1 = _sc_scatter_l1(*y1, src_gg, dst_gg, src_rev, dst_rev)
    y2 = _tc_stage2(acc1, deg_gg, deg_rev,
                    b1_gg.reshape(1, H1), b1_rev.reshape(1, H1), W2_gg, W2_rev)
    zeros2 = jnp.zeros((N_PAD, H2), jnp.float32)
    acc2 = _sc_scatter_l2(*y2, zeros2, src_gg, dst_gg, src_rev, dst_rev)
    g2 = _tc_stage3(acc2, deg_gg, deg_rev,
                    b2_gg.reshape(1, H2), b2_rev.reshape(1, H2))
    ef1, ef2 = _sc_gather_lbl(g2, i0, i1)
    pred = _tc_dot(ef1, ef2)
    return pred[:E_LBL, 0]
